# Initial kernel scaffold; baseline (speedup 1.0000x reference)
import jax
import jax.numpy as jnp
from jax.experimental import pallas as pl


def kernel(x0, adj_rows, adj_cols, adj_vals):
    raise NotImplementedError("write your pallas kernel here")


# speedup vs baseline: 3.6185x; 3.6185x over previous
import dataclasses
import functools

import jax
import jax.numpy as jnp
from jax import lax
from jax.experimental import pallas as pl
from jax.experimental.pallas import tpu as pltpu
from jax.experimental.pallas import tpu_sc as plsc

N = 10000
D = 128
E = 320000
NUM_LAYERS = 3

NC = 2
NS = 16
TILES = NC * NS
C = 128
NCH = 79
EPT = NCH * C
E_PAD = TILES * EPT
LANES = 16

STRIPE = 624
LAST_FLUSH = N - (NS - 1) * STRIPE


def _sc_layer(x, rows3d, cols3d, vals3d):
    mesh = plsc.VectorSubcoreMesh(core_axis_name="c", subcore_axis_name="s")

    cp = pltpu.CompilerParams()
    if "needs_layout_passes" in pltpu.CompilerParams.__dataclass_fields__:
        cp = dataclasses.replace(cp, needs_layout_passes=False)

    @functools.partial(
        pl.kernel,
        out_type=jax.ShapeDtypeStruct((NC, N, D), jnp.float32),
        mesh=mesh,
        compiler_params=cp,
        scratch_types=[
            pltpu.VMEM_SHARED((N, D), jnp.float32),
            pltpu.VMEM((NCH, C), jnp.int32),
            pltpu.VMEM((NCH, C), jnp.int32),
            pltpu.VMEM((NCH, C), jnp.float32),
            pltpu.VMEM((C, D), jnp.float32),
        ],
    )
    def layer(x_hbm, rows_hbm, cols_hbm, vals_hbm, out_hbm,
              acc, rows_v, cols_v, vals_v, g):
        cid = lax.axis_index("c")
        sid = lax.axis_index("s")
        tid = cid * NS + sid

        pltpu.sync_copy(rows_hbm.at[tid], rows_v)
        pltpu.sync_copy(cols_hbm.at[tid], cols_v)
        pltpu.sync_copy(vals_hbm.at[tid], vals_v)

        zv = jnp.zeros((LANES,), jnp.float32)

        @pl.loop(0, C)
        def _(r):
            for k in range(D // LANES):
                g[r, pl.ds(k * LANES, LANES)] = zv

        @pl.loop(0, 5)
        def _(b):
            pltpu.sync_copy(g, acc.at[pl.ds(sid * STRIPE + b * C, C)])

        plsc.subcore_barrier()

        @pl.loop(0, NCH)
        def _(j):
            pltpu.sync_copy(x_hbm.at[cols_v.at[j]], g)

            @pl.loop(0, C)
            def _(i):
                w = plsc.load_gather(
                    vals_v,
                    [jnp.full((LANES,), j, jnp.int32),
                     jnp.full((LANES,), i, jnp.int32)])
                for k in range(D // LANES):
                    sl = pl.ds(k * LANES, LANES)
                    g[i, sl] = g[i, sl] * w

            pltpu.sync_copy(g, acc.at[rows_v.at[j]], add=True)

        plsc.subcore_barrier()

        @pl.when(sid < NS - 1)
        def _():
            pltpu.sync_copy(acc.at[pl.ds(sid * STRIPE, STRIPE)],
                            out_hbm.at[cid, pl.ds(sid * STRIPE, STRIPE)])

        @pl.when(sid == NS - 1)
        def _():
            pltpu.sync_copy(acc.at[pl.ds((NS - 1) * STRIPE, LAST_FLUSH)],
                            out_hbm.at[cid, pl.ds((NS - 1) * STRIPE, LAST_FLUSH)])

    return layer(x, rows3d, cols3d, vals3d)


def _tc_add(parts):
    def body(p_ref, o_ref):
        o_ref[...] = p_ref[0] + p_ref[1]

    return pl.pallas_call(
        body,
        out_shape=jax.ShapeDtypeStruct((N, D), jnp.float32),
        grid=(10,),
        in_specs=[pl.BlockSpec((2, N // 10, D), lambda i: (0, i, 0))],
        out_specs=pl.BlockSpec((N // 10, D), lambda i: (i, 0)),
    )(parts)


@jax.jit
def kernel(x0, adj_rows, adj_cols, adj_vals):
    pad = E_PAD - E
    rows3d = jnp.pad(adj_rows.astype(jnp.int32), (0, pad)).reshape(
        TILES, NCH, C)
    cols3d = jnp.pad(adj_cols.astype(jnp.int32), (0, pad)).reshape(
        TILES, NCH, C)
    vals3d = jnp.pad(adj_vals.astype(jnp.float32), (0, pad)).reshape(
        TILES, NCH, C)

    xs = [x0]
    x = x0
    for _ in range(NUM_LAYERS):
        parts = _sc_layer(x, rows3d, cols3d, vals3d)
        x = _tc_add(parts)
        xs.append(x)
    return tuple(xs)
